# baseline (speedup 1.0000x reference)
import functools

import jax
import jax.numpy as jnp
from jax import lax
from jax.experimental import pallas as pl
from jax.experimental.pallas import tpu as pltpu
from jax.experimental.pallas import tpu_sc as plsc

VOCAB = 1000000
EMBED = 16
BATCH = 16384
LANES = 16
NW = 32
BPW = BATCH // NW
CHUNK = 32
NCHUNK = BPW // CHUNK
GPC = CHUNK // LANES
NBUF = 3


def _sc_call(tid, sid, table, wb, bvec):
    mesh = plsc.VectorSubcoreMesh(core_axis_name="c", subcore_axis_name="s")

    @functools.partial(
        pl.kernel,
        mesh=mesh,
        compiler_params=pltpu.CompilerParams(needs_layout_passes=False),
        out_type=jax.ShapeDtypeStruct((NW, BPW), jnp.float32),
        scratch_types=[
            pltpu.VMEM((BPW,), jnp.int32),
            pltpu.VMEM((BPW,), jnp.int32),
            pltpu.VMEM((CHUNK, 8, EMBED), jnp.float32),
            pltpu.VMEM((CHUNK, 8, EMBED), jnp.float32),
            pltpu.VMEM((CHUNK, 8, EMBED), jnp.float32),
            pltpu.VMEM((EMBED, LANES), jnp.float32),
            pltpu.VMEM((LANES,), jnp.float32),
            pltpu.VMEM((BPW,), jnp.float32),
            pltpu.SemaphoreType.DMA,
            pltpu.SemaphoreType.DMA,
            pltpu.SemaphoreType.DMA,
        ],
    )
    def sc_kernel(tid_hbm, sid_hbm, table_hbm, wb_hbm, b_hbm, out_hbm,
                  tid_v, sid_v, buf0, buf1, buf2, wb_v, b_v, out_v,
                  sem0, sem1, sem2):
        wid = lax.axis_index("s") * 2 + lax.axis_index("c")
        pltpu.sync_copy(tid_hbm.at[wid], tid_v)
        pltpu.sync_copy(sid_hbm.at[wid], sid_v)
        pltpu.sync_copy(wb_hbm, wb_v)
        pltpu.sync_copy(b_hbm, b_v)

        bufs = (buf0, buf1, buf2)
        sems = (sem0, sem1, sem2)

        def fire(c):
            cps = []
            for u in range(CHUNK // LANES):
                tv = tid_v[pl.ds(c * CHUNK + u * LANES, LANES)]
                for k in range(LANES):
                    cps.append(
                        pltpu.async_copy(
                            table_hbm.at[pl.ds(pl.multiple_of(tv[k], 8), 8), :],
                            bufs[c % NBUF].at[u * LANES + k],
                            sems[c % NBUF],
                        )
                    )
            return cps

        wrows = [wb_v[j, :] for j in range(EMBED)]
        bias = b_v[...]
        base_iota = lax.iota(jnp.int32, LANES)

        pend = [fire(0), fire(1)]
        for c in range(NCHUNK):
            if c + 2 < NCHUNK:
                pend.append(fire(c + 2))
            for cp in pend.pop(0):
                cp.wait()
            buf = bufs[c % NBUF]
            for g in range(GPC):
                off = c * CHUNK + g * LANES
                i_ids = base_iota + (g * LANES)
                s_ids = sid_v[pl.ds(off, LANES)]
                acc = bias
                for j in range(EMBED):
                    col = plsc.load_gather(
                        buf, [i_ids, s_ids, jnp.full((LANES,), j, jnp.int32)]
                    )
                    acc = acc + col * wrows[j]
                out_v[pl.ds(off, LANES)] = acc

        pltpu.sync_copy(out_v, out_hbm.at[wid])

    return sc_kernel(tid, sid, table, wb, bvec)


def kernel(x, table, W, b):
    xi = x.reshape(NW, BPW).astype(jnp.int32)
    tid = xi & ~jnp.int32(7)
    sid = xi & 7
    wb = jnp.broadcast_to(
        W.astype(jnp.float32).reshape(EMBED, 1), (EMBED, LANES)
    )
    bvec = jnp.broadcast_to(b.astype(jnp.float32).reshape(1), (LANES,))
    out = _sc_call(tid, sid, table.astype(jnp.float32), wb, bvec)
    return out.reshape(BATCH, 1)


# speedup vs baseline: 1.6276x; 1.6276x over previous
import functools

import jax
import jax.numpy as jnp
from jax import lax
from jax.experimental import pallas as pl
from jax.experimental.pallas import tpu as pltpu
from jax.experimental.pallas import tpu_sc as plsc

VOCAB = 1000000
EMBED = 16
BATCH = 16384
LANES = 16
NW = 32
BPW = BATCH // NW
CHUNK = 32
NCHUNK = BPW // CHUNK
GPC = CHUNK // LANES
NBUF = 3


def _sc_call(tid, sid, table3d, wb, bvec):
    mesh = plsc.VectorSubcoreMesh(core_axis_name="c", subcore_axis_name="s")

    @functools.partial(
        pl.kernel,
        mesh=mesh,
        compiler_params=pltpu.CompilerParams(needs_layout_passes=False),
        out_type=jax.ShapeDtypeStruct((NW, BPW), jnp.float32),
        scratch_types=[
            pltpu.VMEM((BPW,), jnp.int32),
            pltpu.VMEM((BPW,), jnp.int32),
            pltpu.VMEM((CHUNK, 8, EMBED), jnp.float32),
            pltpu.VMEM((CHUNK, 8, EMBED), jnp.float32),
            pltpu.VMEM((CHUNK, 8, EMBED), jnp.float32),
            pltpu.VMEM((EMBED, LANES), jnp.float32),
            pltpu.VMEM((LANES,), jnp.float32),
            pltpu.VMEM((BPW,), jnp.float32),
            pltpu.SemaphoreType.DMA,
            pltpu.SemaphoreType.DMA,
            pltpu.SemaphoreType.DMA,
        ],
    )
    def sc_kernel(tid_hbm, sid_hbm, table_hbm, wb_hbm, b_hbm, out_hbm,
                  tid_v, sid_v, buf0, buf1, buf2, wb_v, b_v, out_v,
                  sem0, sem1, sem2):
        wid = lax.axis_index("s") * 2 + lax.axis_index("c")
        pltpu.sync_copy(tid_hbm.at[wid], tid_v)
        pltpu.sync_copy(sid_hbm.at[wid], sid_v)
        pltpu.sync_copy(wb_hbm, wb_v)
        pltpu.sync_copy(b_hbm, b_v)

        bufs = (buf0, buf1, buf2)
        sems = (sem0, sem1, sem2)

        def fire(c):
            cps = []
            for u in range(CHUNK // LANES):
                tv = tid_v[pl.ds(c * CHUNK + u * LANES, LANES)]
                for k in range(LANES):
                    cps.append(
                        pltpu.async_copy(
                            table_hbm.at[tv[k]],
                            bufs[c % NBUF].at[u * LANES + k],
                            sems[c % NBUF],
                        )
                    )
            return cps

        wrows = [wb_v[j, :] for j in range(EMBED)]
        bias = b_v[...]
        base_iota = lax.iota(jnp.int32, LANES)

        pend = [fire(0), fire(1)]
        for c in range(NCHUNK):
            if c + 2 < NCHUNK:
                pend.append(fire(c + 2))
            for cp in pend.pop(0):
                cp.wait()
            buf = bufs[c % NBUF]
            for g in range(GPC):
                off = c * CHUNK + g * LANES
                i_ids = base_iota + (g * LANES)
                s_ids = sid_v[pl.ds(off, LANES)]
                acc = bias
                for j in range(EMBED):
                    col = plsc.load_gather(
                        buf, [i_ids, s_ids, jnp.full((LANES,), j, jnp.int32)]
                    )
                    acc = acc + col * wrows[j]
                out_v[pl.ds(off, LANES)] = acc

        pltpu.sync_copy(out_v, out_hbm.at[wid])

    return sc_kernel(tid, sid, table3d, wb, bvec)


def kernel(x, table, W, b):
    xi = x.reshape(NW, BPW).astype(jnp.int32)
    tid = xi >> 3
    sid = xi & 7
    table3d = table.astype(jnp.float32).reshape(VOCAB // 8, 8, EMBED)
    wb = jnp.broadcast_to(
        W.astype(jnp.float32).reshape(EMBED, 1), (EMBED, LANES)
    )
    bvec = jnp.broadcast_to(b.astype(jnp.float32).reshape(1), (LANES,))
    out = _sc_call(tid, sid, table3d, wb, bvec)
    return out.reshape(BATCH, 1)
